# baseline (speedup 1.0000x reference)
import jax
import jax.numpy as jnp
from jax.experimental import pallas as pl
from jax.experimental.pallas import tpu as pltpu

_VMEM_LIMIT = 44 * 1024 * 1024
_K1 = 28 * 28


def _fused_body(x_ref,
                a00_ref, a01_ref, a10_ref, a11_ref, b1_ref,
                c00_ref, c01_ref, c10_ref, c11_ref, b2_ref,
                wfc_ref, bfc_ref, o_ref):
    xb = x_ref[...]

    def mm(lhs, w_ref):
        return jnp.dot(lhs, w_ref[...], preferred_element_type=jnp.float32)

    m = mm(xb, a00_ref)
    m = jnp.maximum(m, mm(xb, a01_ref))
    m = jnp.maximum(m, mm(xb, a10_ref))
    m = jnp.maximum(m, mm(xb, a11_ref))
    h1 = jnp.maximum(m + b1_ref[...], 0.0).astype(jnp.bfloat16)

    m2 = mm(h1, c00_ref)
    m2 = jnp.maximum(m2, mm(h1, c01_ref))
    m2 = jnp.maximum(m2, mm(h1, c10_ref))
    m2 = jnp.maximum(m2, mm(h1, c11_ref))
    h2 = jnp.maximum(m2 + b2_ref[...], 0.0).astype(jnp.bfloat16)

    logits = mm(h2, wfc_ref) + bfc_ref[...]
    mx = jnp.max(logits, axis=-1, keepdims=True)
    s = logits - mx
    lse = jnp.log(jnp.sum(jnp.exp(s), axis=-1, keepdims=True))
    o_ref[...] = (s - lse)[:, :10]


def kernel(x, a00, a01, a10, a11, b1, c00, c01, c10, c11, b2, wfc, bfc):
    n = x.shape[0]
    x2d = x.reshape(n, _K1).astype(jnp.bfloat16)

    tb = 1024 if n >= 1024 else 8 * pl.cdiv(n, 8)
    grid = pl.cdiv(n, tb)
    n_pad = grid * tb
    if n_pad != n:
        x2d = jnp.pad(x2d, ((0, n_pad - n), (0, 0)))

    def const_spec(arr):
        return pl.BlockSpec(arr.shape, lambda i: (0, 0))

    weights = [a00, a01, a10, a11, b1, c00, c01, c10, c11, b2, wfc, bfc]
    out = pl.pallas_call(
        _fused_body,
        out_shape=jax.ShapeDtypeStruct((n_pad, 10), jnp.float32),
        grid=(grid,),
        in_specs=[pl.BlockSpec((tb, _K1), lambda i: (i, 0))]
                 + [const_spec(w) for w in weights],
        out_specs=pl.BlockSpec((tb, 10), lambda i: (i, 0)),
        compiler_params=pltpu.CompilerParams(
            dimension_semantics=("parallel",),
            vmem_limit_bytes=_VMEM_LIMIT),
    )(x2d, *weights)
    return out[:n]


# speedup vs baseline: 1.0183x; 1.0183x over previous
import jax
import jax.numpy as jnp
from jax.experimental import pallas as pl
from jax.experimental.pallas import tpu as pltpu

_VMEM_LIMIT = 44 * 1024 * 1024
_K1 = 28 * 28
_H1, _H1P = 1440, 1536
_H2, _H2P = 320, 384


def _fused_body(x_ref, al_ref, ar_ref, b1_ref, c_ref, b2_ref,
                wfc_ref, bfc_ref, o_ref):
    xb = x_ref[...]

    zl = jnp.dot(xb, al_ref[...], preferred_element_type=jnp.float32)
    ml = jnp.maximum(zl[:, 0:_H1], zl[:, _H1P:_H1P + _H1])
    zr = jnp.dot(xb, ar_ref[...], preferred_element_type=jnp.float32)
    mr = jnp.maximum(zr[:, 0:_H1], zr[:, _H1P:_H1P + _H1])
    h1 = jnp.maximum(jnp.maximum(ml, mr) + b1_ref[...],
                     0.0).astype(jnp.bfloat16)

    z2 = jnp.dot(h1, c_ref[...], preferred_element_type=jnp.float32)
    m2 = jnp.maximum(
        jnp.maximum(z2[:, 0:_H2], z2[:, _H2P:_H2P + _H2]),
        jnp.maximum(z2[:, 2 * _H2P:2 * _H2P + _H2],
                    z2[:, 3 * _H2P:3 * _H2P + _H2]))
    h2 = jnp.maximum(m2 + b2_ref[...], 0.0).astype(jnp.bfloat16)

    logits = jnp.dot(h2, wfc_ref[...],
                     preferred_element_type=jnp.float32) + bfc_ref[...]
    mx = jnp.max(logits, axis=-1, keepdims=True)
    s = logits - mx
    lse = jnp.log(jnp.sum(jnp.exp(s), axis=-1, keepdims=True))
    o_ref[...] = (s - lse)[:, :10]


def _pad_cat(mats, width):
    return jnp.concatenate(
        [jnp.pad(m, ((0, 0), (0, width - m.shape[1]))) for m in mats], axis=1)


def kernel(x, a00, a01, a10, a11, b1, c00, c01, c10, c11, b2, wfc, bfc):
    n = x.shape[0]
    x2d = x.reshape(n, _K1).astype(jnp.bfloat16)
    a_l = _pad_cat([a00, a01], _H1P)
    a_r = _pad_cat([a10, a11], _H1P)
    c_cat = _pad_cat([c00, c01, c10, c11], _H2P)

    tb = 1024 if n >= 1024 else 8 * pl.cdiv(n, 8)
    grid = pl.cdiv(n, tb)
    n_pad = grid * tb
    if n_pad != n:
        x2d = jnp.pad(x2d, ((0, n_pad - n), (0, 0)))

    def const_spec(arr):
        return pl.BlockSpec(arr.shape, lambda i: (0, 0))

    weights = [a_l, a_r, b1, c_cat, b2, wfc, bfc]
    out = pl.pallas_call(
        _fused_body,
        out_shape=jax.ShapeDtypeStruct((n_pad, 10), jnp.float32),
        grid=(grid,),
        in_specs=[pl.BlockSpec((tb, _K1), lambda i: (i, 0))]
                 + [const_spec(w) for w in weights],
        out_specs=pl.BlockSpec((tb, 10), lambda i: (i, 0)),
        compiler_params=pltpu.CompilerParams(
            dimension_semantics=("parallel",),
            vmem_limit_bytes=_VMEM_LIMIT),
    )(x2d, *weights)
    return out[:n]
